# Initial kernel scaffold; baseline (speedup 1.0000x reference)
import jax
import jax.numpy as jnp
from jax.experimental import pallas as pl


def kernel(surface_pos, init_ligand_pos, batch_surface, batch_ligand, time, params):
    raise NotImplementedError("write your pallas kernel here")


# speedup vs baseline: 2.6622x; 2.6622x over previous
import functools

import jax
import jax.numpy as jnp
from jax import lax
from jax.experimental import pallas as pl
from jax.experimental.pallas import tpu as pltpu
from jax.experimental.pallas import tpu_sc as plsc

S = 8000
LG = 2000
F = 64
NC, NS = 2, 16
NW = NC * NS
CPT = 8192
CAP = NW * CPT
SPAD = 8192
LPAD = 2048
EBLK = 2048
STS = 400
GCH = 128

def _mesh():
    return plsc.VectorSubcoreMesh(core_axis_name="c", subcore_axis_name="s")


def _sc_params():
    return pltpu.CompilerParams(use_tc_tiling_on_sc=False)


def _lig_prologue_body(lp_ref, t_ref, tw1, tb1, tw2, tb2, wl, bl, wg, bg, wb,
                       out_ref):
    t = t_ref[...]
    half = 32
    k = lax.broadcasted_iota(jnp.int32, (1, half), 1).astype(jnp.float32)
    freqs = jnp.exp(-jnp.log(10000.0) / (half - 1) * k)
    a = t * freqs
    ht = jnp.concatenate([jnp.sin(a), jnp.cos(a)], axis=1)
    x = jnp.dot(ht, tw1[...], preferred_element_type=jnp.float32) + tb1[...]
    c = 0.7978845608028654
    g = 0.5 * x * (1.0 + jnp.tanh(c * (x + 0.044715 * x * x * x)))
    ht = jnp.dot(g, tw2[...], preferred_element_type=jnp.float32) + tb2[...]
    lp = lp_ref[...]
    base = jnp.dot(lp, wl[...], preferred_element_type=jnp.float32) + bl[...]
    gate = jax.nn.sigmoid(
        jnp.dot(ht, wg[...], preferred_element_type=jnp.float32) + bg[...])
    out_ref[...] = base * gate + jnp.dot(ht, wb[...],
                                         preferred_element_type=jnp.float32)


def _lig_prologue(lp, t, p):
    tm, c = p['time_mlp'], p['csl']
    full = lambda s: pl.BlockSpec(s, lambda: (0,) * len(s))
    args = (lp, t,
            tm['w1'], tm['b1'].reshape(1, -1), tm['w2'], tm['b2'].reshape(1, -1),
            c['wl'], c['bl'].reshape(1, -1), c['wg'], c['bg'].reshape(1, -1),
            c['wb'])
    return pl.pallas_call(
        _lig_prologue_body,
        out_shape=jax.ShapeDtypeStruct((LG, F), jnp.float32),
        in_specs=[full(a.shape) for a in args],
        out_specs=full((LG, F)),
    )(*args)


def _ln(x, g, b):
    m = jnp.mean(x, axis=-1, keepdims=True)
    v = jnp.mean((x - m) ** 2, axis=-1, keepdims=True)
    return (x - m) * jax.lax.rsqrt(v + 1e-5) * g + b


def _surf_mask_body(sp_ref, bs_ref, lpt_ref, bl_ref, sw, sb,
                    nw1, nb1, nw2, nb2, ng, nbl,
                    mask_ref, hs0_ref, hs1_ref, hs2_ref):
    sp = sp_ref[...]
    d2 = jnp.zeros((STS, LG), jnp.float32)
    for ci in range(3):
        diff = sp[:, ci:ci + 1] - lpt_ref[ci:ci + 1, :]
        d2 = d2 + diff * diff
    same = bs_ref[...] == bl_ref[...]
    mask_ref[...] = jnp.where(same & (d2 < 9.0), jnp.int32(1), jnp.int32(0))

    hs = jnp.dot(sp, sw[...], preferred_element_type=jnp.float32) + sb[...]
    hs0_ref[...] = hs
    outs = (hs1_ref, hs2_ref)
    for li in range(2):
        up = jnp.maximum(
            jnp.dot(hs, nw1[li], preferred_element_type=jnp.float32) + nb1[li],
            0.0)
        up = jnp.dot(up, nw2[li], preferred_element_type=jnp.float32) + nb2[li]
        hs = hs + _ln(up, ng[li], nbl[li])
        outs[li][...] = hs


def _surf_mask(sp, bs, lp, bl, p):
    nw1 = jnp.stack([cv['node']['w1'][:F] for cv in p['convs'][:2]])
    nb1 = jnp.stack([cv['node']['b1'].reshape(1, -1) for cv in p['convs'][:2]])
    nw2 = jnp.stack([cv['node']['w2'] for cv in p['convs'][:2]])
    nb2 = jnp.stack([cv['node']['b2'].reshape(1, -1) for cv in p['convs'][:2]])
    ng = jnp.stack([cv['node']['ln_g'].reshape(1, -1) for cv in p['convs'][:2]])
    nbl = jnp.stack([cv['node']['ln_b'].reshape(1, -1) for cv in p['convs'][:2]])
    grid = S // STS
    tile = lambda s: pl.BlockSpec(s, lambda i: (i,) + (0,) * (len(s) - 1))
    full = lambda s: pl.BlockSpec(s, lambda i: (0,) * len(s))
    args = (sp, bs.reshape(S, 1), lp.T, bl.reshape(1, LG),
            p['surf_enc']['w'], p['surf_enc']['b'].reshape(1, -1),
            nw1, nb1, nw2, nb2, ng, nbl)
    in_specs = [tile((STS, 3)), tile((STS, 1)), full((3, LG)), full((1, LG))]
    in_specs += [full(a.shape) for a in args[4:]]
    return pl.pallas_call(
        _surf_mask_body,
        grid=(grid,),
        out_shape=[jax.ShapeDtypeStruct((S, LG), jnp.int32)] +
                  [jax.ShapeDtypeStruct((S, F), jnp.float32)] * 3,
        in_specs=in_specs,
        out_specs=[tile((STS, LG))] + [tile((STS, F))] * 3,
    )(*args)


def _sc_gather(table, idx):
    D = table.shape[1]

    @functools.partial(
        pl.kernel, mesh=_mesh(), compiler_params=_sc_params(),
        out_type=jax.ShapeDtypeStruct((CAP, D), jnp.float32),
        scratch_types=[pltpu.VMEM((GCH,), jnp.int32),
                       pltpu.VMEM((GCH, D), jnp.float32),
                       pltpu.SemaphoreType.DMA])
    def k(tab_hbm, idx_hbm, out_hbm, idx_v, rows_v, sem):
        wid = lax.axis_index("s") * NC + lax.axis_index("c")
        base = wid * CPT

        @pl.loop(0, CPT, step=GCH)
        def _(off):
            pltpu.sync_copy(idx_hbm.at[pl.ds(base + off, GCH)], idx_v)
            pltpu.async_copy(tab_hbm.at[idx_v], rows_v, sem).wait()
            pltpu.sync_copy(rows_v, out_hbm.at[pl.ds(base + off, GCH)])

    return k(table, idx)


def _sc_scatter_add(vals, dst, zeros):
    @functools.partial(
        pl.kernel, mesh=_mesh(), compiler_params=_sc_params(),
        out_type=jax.ShapeDtypeStruct((NC, LPAD, F), jnp.float32),
        scratch_types=[pltpu.VMEM((GCH,), jnp.int32),
                       pltpu.VMEM((GCH, F), jnp.float32),
                       pltpu.VMEM_SHARED((LPAD, F), jnp.float32),
                       pltpu.SemaphoreType.DMA])
    def k(v_hbm, d_hbm, z_hbm, out_hbm, idx_v, rows_v, acc_sh, sem):
        cid = lax.axis_index("c")
        sid = lax.axis_index("s")
        wid = sid * NC + cid
        base = wid * CPT
        stripe = LPAD // NS
        pltpu.sync_copy(z_hbm.at[pl.ds(sid * stripe, stripe)],
                        acc_sh.at[pl.ds(sid * stripe, stripe)])
        plsc.subcore_barrier()

        @pl.loop(0, CPT, step=GCH)
        def _(off):
            pltpu.sync_copy(d_hbm.at[pl.ds(base + off, GCH)], idx_v)
            pltpu.sync_copy(v_hbm.at[pl.ds(base + off, GCH)], rows_v)
            pltpu.sync_copy(rows_v, acc_sh.at[idx_v], add=True)

        plsc.subcore_barrier()
        pltpu.sync_copy(acc_sh.at[pl.ds(sid * stripe, stripe)],
                        out_hbm.at[cid].at[pl.ds(sid * stripe, stripe)])

    return k(vals, dst, zeros)


def _edge_init_body(sps_ref, lps_ref, emw, off, out_ref):
    ev = sps_ref[:, 0:3] - lps_ref[:, 0:3]
    nrm = jnp.sqrt(jnp.sum(ev * ev, axis=1, keepdims=True))
    v = ev / (nrm + 1e-7)
    o = off[...]
    coeff = -0.5 / ((10.0 / 18.0) ** 2)
    sca = jnp.exp(coeff * (nrm - o) ** 2)
    em = emw[...]
    parts = [sca] + [v[:, ci:ci + 1] * em for ci in range(3)]
    out_ref[...] = jnp.concatenate(parts, axis=1)


def _edge_init(sps, lps, p):
    full = lambda s: pl.BlockSpec(s, lambda i: (0,) * len(s))
    tile = lambda s: pl.BlockSpec(s, lambda i: (i,) + (0,) * (len(s) - 1))
    return pl.pallas_call(
        _edge_init_body,
        grid=(CAP // EBLK,),
        out_shape=jax.ShapeDtypeStruct((CAP, F), jnp.float32),
        in_specs=[tile((EBLK, 16)), tile((EBLK, 16)),
                  full((1, 15)), full((1, 19))],
        out_specs=tile((EBLK, F)),
    )(sps, lps, p['edge_map_w'], p['gs_offset'].reshape(1, 19))


def _edge_mlp_body(gs_ref, gl_ref, he_ref, w1, b1, w2, b2, g, b, out_ref):
    w = w1[...]
    x = (jnp.dot(gs_ref[...], w[0:F], preferred_element_type=jnp.float32) +
         jnp.dot(gl_ref[...], w[F:2 * F], preferred_element_type=jnp.float32) +
         jnp.dot(he_ref[...], w[2 * F:3 * F], preferred_element_type=jnp.float32)
         + b1[...])
    x = jnp.maximum(x, 0.0)
    x = jnp.dot(x, w2[...], preferred_element_type=jnp.float32) + b2[...]
    out_ref[...] = he_ref[...] + _ln(x, g[...], b[...])


def _edge_mlp(gs, gl, he, cv):
    full = lambda s: pl.BlockSpec(s, lambda i: (0,) * len(s))
    tile = lambda s: pl.BlockSpec(s, lambda i: (i,) + (0,) * (len(s) - 1))
    e = cv['edge']
    return pl.pallas_call(
        _edge_mlp_body,
        grid=(CAP // EBLK,),
        out_shape=jax.ShapeDtypeStruct((CAP, F), jnp.float32),
        in_specs=[tile((EBLK, F))] * 3 + [
            full((192, 128)), full((1, 128)), full((128, F)), full((1, F)),
            full((1, F)), full((1, F))],
        out_specs=tile((EBLK, F)),
    )(gs, gl, he, e['w1'], e['b1'].reshape(1, -1), e['w2'],
      e['b2'].reshape(1, -1), e['ln_g'].reshape(1, -1), e['ln_b'].reshape(1, -1))


def _lig_node_body(hl_ref, agg_ref, w1, b1, w2, b2, g, b, out_ref):
    agg = agg_ref[0, 0:LG, :] + agg_ref[1, 0:LG, :]
    hl = hl_ref[...]
    w = w1[...]
    x = (jnp.dot(hl, w[0:F], preferred_element_type=jnp.float32) +
         jnp.dot(agg, w[F:2 * F], preferred_element_type=jnp.float32) + b1[...])
    x = jnp.maximum(x, 0.0)
    x = jnp.dot(x, w2[...], preferred_element_type=jnp.float32) + b2[...]
    out_ref[...] = hl + _ln(x, g[...], b[...])


def _lig_node(hl, agg2, cv):
    full = lambda s: pl.BlockSpec(s, lambda: (0,) * len(s))
    n = cv['node']
    return pl.pallas_call(
        _lig_node_body,
        out_shape=jax.ShapeDtypeStruct((LG, F), jnp.float32),
        in_specs=[full((LG, F)), full((NC, LPAD, F)),
                  full((128, 128)), full((1, 128)), full((128, F)),
                  full((1, F)), full((1, F)), full((1, F))],
        out_specs=full((LG, F)),
    )(hl, agg2, n['w1'], n['b1'].reshape(1, -1), n['w2'],
      n['b2'].reshape(1, -1), n['ln_g'].reshape(1, -1),
      n['ln_b'].reshape(1, -1))


def _pos_out_body(hl_ref, lp_ref, w1, b1, w2, b2, out_ref):
    x = jnp.dot(hl_ref[...], w1[...], preferred_element_type=jnp.float32) + b1[...]
    x = jnp.maximum(x, 0.0)
    x = jnp.dot(x, w2[...], preferred_element_type=jnp.float32) + b2[...]
    out_ref[...] = x + lp_ref[...]


def _pos_out(hl, lp, p):
    full = lambda s: pl.BlockSpec(s, lambda: (0,) * len(s))
    m = p['pos_mlp']
    return pl.pallas_call(
        _pos_out_body,
        out_shape=jax.ShapeDtypeStruct((LG, 3), jnp.float32),
        in_specs=[full((LG, F)), full((LG, 3)), full((F, F)), full((1, F)),
                  full((F, 3)), full((1, 3))],
        out_specs=full((LG, 3)),
    )(hl, lp, m['w1'], m['b1'].reshape(1, -1), m['w2'], m['b2'].reshape(1, -1))


def kernel(surface_pos, init_ligand_pos, batch_surface, batch_ligand, time,
           params):
    p = params
    hl0 = _lig_prologue(init_ligand_pos, time, p)
    mask, hs0, hs1, hs2 = _surf_mask(surface_pos, batch_surface,
                                     init_ligand_pos, batch_ligand, p)

    flat = jnp.nonzero(mask.reshape(-1), size=CAP,
                       fill_value=S * LG)[0].astype(jnp.int32)
    valid = flat < S * LG
    src = jnp.where(valid, flat // LG, S).astype(jnp.int32)
    dst = jnp.where(valid, flat - (flat // LG) * LG, LG).astype(jnp.int32)

    pad_rows = lambda x, n: jnp.pad(x, ((0, n - x.shape[0]), (0, 0)))
    sp_pad = jnp.pad(surface_pos, ((0, SPAD - S), (0, 13)))
    lp_pad = jnp.pad(init_ligand_pos, ((0, LPAD - LG), (0, 13)))
    sps = _sc_gather(sp_pad, src)
    lps = _sc_gather(lp_pad, dst)
    he = _edge_init(sps, lps, p)

    zeros = jnp.zeros((LPAD, F), jnp.float32)
    hs_traj = [hs0, hs1, hs2]
    hl = hl0
    for li, cv in enumerate(p['convs']):
        gs = _sc_gather(pad_rows(hs_traj[li], SPAD), src)
        gl = _sc_gather(pad_rows(hl, LPAD), dst)
        he = _edge_mlp(gs, gl, he, cv)
        agg2 = _sc_scatter_add(he, dst, zeros)
        hl = _lig_node(hl, agg2, cv)

    return _pos_out(hl, init_ligand_pos, p)


# speedup vs baseline: 6.6251x; 2.4885x over previous
import functools

import jax
import jax.numpy as jnp
from jax import lax
from jax.experimental import pallas as pl
from jax.experimental.pallas import tpu as pltpu
from jax.experimental.pallas import tpu_sc as plsc

S = 8000
LG = 2000
F = 64
NC, NS = 2, 16
NW = NC * NS
CPT = 5120
CAP = NW * CPT
SPAD = 8192
LPAD = 2048
EBLK = 2048
STS = 400
GCH = 128

def _mesh():
    return plsc.VectorSubcoreMesh(core_axis_name="c", subcore_axis_name="s")


def _sc_params():
    return pltpu.CompilerParams(use_tc_tiling_on_sc=False)


def _lig_prologue_body(lp_ref, t_ref, tw1, tb1, tw2, tb2, wl, bl, wg, bg, wb,
                       out_ref):
    t = t_ref[...]
    half = 32
    k = lax.broadcasted_iota(jnp.int32, (1, half), 1).astype(jnp.float32)
    freqs = jnp.exp(-jnp.log(10000.0) / (half - 1) * k)
    a = t * freqs
    ht = jnp.concatenate([jnp.sin(a), jnp.cos(a)], axis=1)
    x = jnp.dot(ht, tw1[...], preferred_element_type=jnp.float32) + tb1[...]
    c = 0.7978845608028654
    g = 0.5 * x * (1.0 + jnp.tanh(c * (x + 0.044715 * x * x * x)))
    ht = jnp.dot(g, tw2[...], preferred_element_type=jnp.float32) + tb2[...]
    lp = lp_ref[...]
    base = jnp.dot(lp, wl[...], preferred_element_type=jnp.float32) + bl[...]
    gate = jax.nn.sigmoid(
        jnp.dot(ht, wg[...], preferred_element_type=jnp.float32) + bg[...])
    out_ref[...] = base * gate + jnp.dot(ht, wb[...],
                                         preferred_element_type=jnp.float32)


def _lig_prologue(lp, t, p):
    tm, c = p['time_mlp'], p['csl']
    full = lambda s: pl.BlockSpec(s, lambda: (0,) * len(s))
    args = (lp, t,
            tm['w1'], tm['b1'].reshape(1, -1), tm['w2'], tm['b2'].reshape(1, -1),
            c['wl'], c['bl'].reshape(1, -1), c['wg'], c['bg'].reshape(1, -1),
            c['wb'])
    return pl.pallas_call(
        _lig_prologue_body,
        out_shape=jax.ShapeDtypeStruct((LG, F), jnp.float32),
        in_specs=[full(a.shape) for a in args],
        out_specs=full((LG, F)),
    )(*args)


def _ln(x, g, b):
    m = jnp.mean(x, axis=-1, keepdims=True)
    v = jnp.mean((x - m) ** 2, axis=-1, keepdims=True)
    return (x - m) * jax.lax.rsqrt(v + 1e-5) * g + b


def _surf_mask_body(sp_ref, bs_ref, lpt_ref, bl_ref, sw, sb,
                    nw1, nb1, nw2, nb2, ng, nbl,
                    mask_ref, hs0_ref, hs1_ref, hs2_ref):
    sp = sp_ref[...]
    d2 = jnp.zeros((STS, LG), jnp.float32)
    for ci in range(3):
        diff = sp[:, ci:ci + 1] - lpt_ref[ci:ci + 1, :]
        d2 = d2 + diff * diff
    same = bs_ref[...] == bl_ref[...]
    mask_ref[...] = jnp.where(same & (d2 < 9.0), jnp.int32(1), jnp.int32(0))

    hs = jnp.dot(sp, sw[...], preferred_element_type=jnp.float32) + sb[...]
    hs0_ref[...] = hs
    outs = (hs1_ref, hs2_ref)
    for li in range(2):
        up = jnp.maximum(
            jnp.dot(hs, nw1[li], preferred_element_type=jnp.float32) + nb1[li],
            0.0)
        up = jnp.dot(up, nw2[li], preferred_element_type=jnp.float32) + nb2[li]
        hs = hs + _ln(up, ng[li], nbl[li])
        outs[li][...] = hs


def _surf_mask(sp, bs, lp, bl, p):
    nw1 = jnp.stack([cv['node']['w1'][:F] for cv in p['convs'][:2]])
    nb1 = jnp.stack([cv['node']['b1'].reshape(1, -1) for cv in p['convs'][:2]])
    nw2 = jnp.stack([cv['node']['w2'] for cv in p['convs'][:2]])
    nb2 = jnp.stack([cv['node']['b2'].reshape(1, -1) for cv in p['convs'][:2]])
    ng = jnp.stack([cv['node']['ln_g'].reshape(1, -1) for cv in p['convs'][:2]])
    nbl = jnp.stack([cv['node']['ln_b'].reshape(1, -1) for cv in p['convs'][:2]])
    grid = S // STS
    tile = lambda s: pl.BlockSpec(s, lambda i: (i,) + (0,) * (len(s) - 1))
    full = lambda s: pl.BlockSpec(s, lambda i: (0,) * len(s))
    args = (sp, bs.reshape(S, 1), lp.T, bl.reshape(1, LG),
            p['surf_enc']['w'], p['surf_enc']['b'].reshape(1, -1),
            nw1, nb1, nw2, nb2, ng, nbl)
    in_specs = [tile((STS, 3)), tile((STS, 1)), full((3, LG)), full((1, LG))]
    in_specs += [full(a.shape) for a in args[4:]]
    return pl.pallas_call(
        _surf_mask_body,
        grid=(grid,),
        out_shape=[jax.ShapeDtypeStruct((S, LG), jnp.int32)] +
                  [jax.ShapeDtypeStruct((S, F), jnp.float32)] * 3,
        in_specs=in_specs,
        out_specs=[tile((STS, LG))] + [tile((STS, F))] * 3,
    )(*args)


def _sc_gather(table, idx3):
    D = table.shape[1]
    R = 131072 // (4 * D)
    iters = CPT // R
    per = R // GCH

    @functools.partial(
        pl.kernel, mesh=_mesh(), compiler_params=_sc_params(),
        out_type=jax.ShapeDtypeStruct((CAP, D), jnp.float32),
        scratch_types=[pltpu.VMEM((CPT // GCH, GCH), jnp.int32),
                       pltpu.VMEM((2, R, D), jnp.float32),
                       pltpu.SemaphoreType.DMA, pltpu.SemaphoreType.DMA,
                       pltpu.SemaphoreType.DMA, pltpu.SemaphoreType.DMA])
    def k(tab_hbm, idx_hbm, out_hbm, idx_v, rows_v, g0, g1, w0, w1):
        wid = lax.axis_index("s") * NC + lax.axis_index("c")
        base = wid * CPT
        gsem = (g0, g1)
        wsem = (w0, w1)
        pltpu.sync_copy(idx_hbm.at[wid], idx_v)

        def fire(i, b):
            hs = []
            for j in range(per):
                hs.append(pltpu.async_copy(
                    tab_hbm.at[idx_v.at[i * per + j]],
                    rows_v.at[b, pl.ds(j * GCH, GCH)], gsem[b]))
            return hs

        gh = {0: fire(0, 0)}
        wh = {}
        for i in range(iters):
            b = i % 2
            if i + 1 < iters:
                if i >= 1:
                    for h in wh.pop(1 - b):
                        h.wait()
                gh[1 - b] = fire(i + 1, 1 - b)
            for h in gh.pop(b):
                h.wait()
            wh[b] = [pltpu.async_copy(
                rows_v.at[b], out_hbm.at[pl.ds(base + i * R, R)], wsem[b])]
        for hs in wh.values():
            for h in hs:
                h.wait()

    return k(table, idx3)


def _sc_scatter_add(vals, dst3, zeros):
    R = 512
    iters = CPT // R
    per = R // GCH

    @functools.partial(
        pl.kernel, mesh=_mesh(), compiler_params=_sc_params(),
        out_type=jax.ShapeDtypeStruct((NC, LPAD, F), jnp.float32),
        scratch_types=[pltpu.VMEM((CPT // GCH, GCH), jnp.int32),
                       pltpu.VMEM((2, R, F), jnp.float32),
                       pltpu.VMEM_SHARED((LPAD, F), jnp.float32),
                       pltpu.SemaphoreType.DMA, pltpu.SemaphoreType.DMA])
    def k(v_hbm, d_hbm, z_hbm, out_hbm, idx_v, rows_v, acc_sh, l0, l1):
        cid = lax.axis_index("c")
        sid = lax.axis_index("s")
        wid = sid * NC + cid
        base = wid * CPT
        stripe = LPAD // NS
        lsem = (l0, l1)
        pltpu.sync_copy(z_hbm.at[pl.ds(sid * stripe, stripe)],
                        acc_sh.at[pl.ds(sid * stripe, stripe)])
        pltpu.sync_copy(d_hbm.at[wid], idx_v)
        plsc.subcore_barrier()

        def fire(i, b):
            return pltpu.async_copy(
                v_hbm.at[pl.ds(base + i * R, R)], rows_v.at[b], lsem[b])

        h = {0: fire(0, 0)}
        for i in range(iters):
            b = i % 2
            if i + 1 < iters:
                h[1 - b] = fire(i + 1, 1 - b)
            h.pop(b).wait()
            for j in range(per):
                pltpu.sync_copy(rows_v.at[b, pl.ds(j * GCH, GCH)],
                                acc_sh.at[idx_v.at[i * per + j]], add=True)

        plsc.subcore_barrier()
        pltpu.sync_copy(acc_sh.at[pl.ds(sid * stripe, stripe)],
                        out_hbm.at[cid].at[pl.ds(sid * stripe, stripe)])

    return k(vals, dst3, zeros)


def _edge_mlp_common(gs, gl, he, w1, b1, w2, b2, g, b):
    w = w1[...]
    x = (jnp.dot(gs, w[0:F], preferred_element_type=jnp.float32) +
         jnp.dot(gl, w[F:2 * F], preferred_element_type=jnp.float32) +
         jnp.dot(he, w[2 * F:3 * F], preferred_element_type=jnp.float32)
         + b1[...])
    x = jnp.maximum(x, 0.0)
    x = jnp.dot(x, w2[...], preferred_element_type=jnp.float32) + b2[...]
    return he + _ln(x, g[...], b[...])


def _edge_mlp0_body(gsurf_ref, glig_ref, w1, b1, w2, b2, g, b, emw, off,
                    out_ref):
    ev = gsurf_ref[:, 0:3] - glig_ref[:, 0:3]
    nrm = jnp.sqrt(jnp.sum(ev * ev, axis=1, keepdims=True))
    v = ev / (nrm + 1e-7)
    coeff = -0.5 / ((10.0 / 18.0) ** 2)
    sca = jnp.exp(coeff * (nrm - off[...]) ** 2)
    em = emw[...]
    he0 = jnp.concatenate([sca] + [v[:, ci:ci + 1] * em for ci in range(3)],
                          axis=1)
    gs = gsurf_ref[:, 16:16 + F]
    gl = glig_ref[:, 16:16 + F]
    out_ref[...] = _edge_mlp_common(gs, gl, he0, w1, b1, w2, b2, g, b)


def _edge_mlp0(gsurf, glig, cv, p):
    full = lambda s: pl.BlockSpec(s, lambda i: (0,) * len(s))
    tile = lambda s: pl.BlockSpec(s, lambda i: (i,) + (0,) * (len(s) - 1))
    e = cv['edge']
    return pl.pallas_call(
        _edge_mlp0_body,
        grid=(CAP // EBLK,),
        out_shape=jax.ShapeDtypeStruct((CAP, F), jnp.float32),
        in_specs=[tile((EBLK, 256)), tile((EBLK, 128)),
                  full((192, 128)), full((1, 128)), full((128, F)),
                  full((1, F)), full((1, F)), full((1, F)),
                  full((1, 15)), full((1, 19))],
        out_specs=tile((EBLK, F)),
    )(gsurf, glig, e['w1'], e['b1'].reshape(1, -1), e['w2'],
      e['b2'].reshape(1, -1), e['ln_g'].reshape(1, -1),
      e['ln_b'].reshape(1, -1), p['edge_map_w'],
      p['gs_offset'].reshape(1, 19))


def _edge_mlpN_body(col, gsurf_ref, ghl_ref, he_ref, w1, b1, w2, b2, g, b,
                    out_ref):
    gs = gsurf_ref[:, col:col + F]
    out_ref[...] = _edge_mlp_common(gs, ghl_ref[...], he_ref[...],
                                    w1, b1, w2, b2, g, b)


def _edge_mlpN(gsurf, ghl, he, cv, col):
    full = lambda s: pl.BlockSpec(s, lambda i: (0,) * len(s))
    tile = lambda s: pl.BlockSpec(s, lambda i: (i,) + (0,) * (len(s) - 1))
    e = cv['edge']
    return pl.pallas_call(
        functools.partial(_edge_mlpN_body, col),
        grid=(CAP // EBLK,),
        out_shape=jax.ShapeDtypeStruct((CAP, F), jnp.float32),
        in_specs=[tile((EBLK, 256)), tile((EBLK, F)), tile((EBLK, F)),
                  full((192, 128)), full((1, 128)), full((128, F)),
                  full((1, F)), full((1, F)), full((1, F))],
        out_specs=tile((EBLK, F)),
    )(gsurf, ghl, he, e['w1'], e['b1'].reshape(1, -1), e['w2'],
      e['b2'].reshape(1, -1), e['ln_g'].reshape(1, -1),
      e['ln_b'].reshape(1, -1))


def _lig_node_body(hl_ref, agg_ref, w1, b1, w2, b2, g, b, out_ref):
    agg = agg_ref[0, 0:LG, :] + agg_ref[1, 0:LG, :]
    hl = hl_ref[...]
    w = w1[...]
    x = (jnp.dot(hl, w[0:F], preferred_element_type=jnp.float32) +
         jnp.dot(agg, w[F:2 * F], preferred_element_type=jnp.float32) + b1[...])
    x = jnp.maximum(x, 0.0)
    x = jnp.dot(x, w2[...], preferred_element_type=jnp.float32) + b2[...]
    out_ref[...] = hl + _ln(x, g[...], b[...])


def _lig_node(hl, agg2, cv):
    full = lambda s: pl.BlockSpec(s, lambda: (0,) * len(s))
    n = cv['node']
    return pl.pallas_call(
        _lig_node_body,
        out_shape=jax.ShapeDtypeStruct((LG, F), jnp.float32),
        in_specs=[full((LG, F)), full((NC, LPAD, F)),
                  full((128, 128)), full((1, 128)), full((128, F)),
                  full((1, F)), full((1, F)), full((1, F))],
        out_specs=full((LG, F)),
    )(hl, agg2, n['w1'], n['b1'].reshape(1, -1), n['w2'],
      n['b2'].reshape(1, -1), n['ln_g'].reshape(1, -1),
      n['ln_b'].reshape(1, -1))


def _pos_out_body(hl_ref, lp_ref, w1, b1, w2, b2, out_ref):
    x = jnp.dot(hl_ref[...], w1[...], preferred_element_type=jnp.float32) + b1[...]
    x = jnp.maximum(x, 0.0)
    x = jnp.dot(x, w2[...], preferred_element_type=jnp.float32) + b2[...]
    out_ref[...] = x + lp_ref[...]


def _pos_out(hl, lp, p):
    full = lambda s: pl.BlockSpec(s, lambda: (0,) * len(s))
    m = p['pos_mlp']
    return pl.pallas_call(
        _pos_out_body,
        out_shape=jax.ShapeDtypeStruct((LG, 3), jnp.float32),
        in_specs=[full((LG, F)), full((LG, 3)), full((F, F)), full((1, F)),
                  full((F, 3)), full((1, 3))],
        out_specs=full((LG, 3)),
    )(hl, lp, m['w1'], m['b1'].reshape(1, -1), m['w2'], m['b2'].reshape(1, -1))


def kernel(surface_pos, init_ligand_pos, batch_surface, batch_ligand, time,
           params):
    p = params
    hl0 = _lig_prologue(init_ligand_pos, time, p)
    mask, hs0, hs1, hs2 = _surf_mask(surface_pos, batch_surface,
                                     init_ligand_pos, batch_ligand, p)

    flat = jnp.nonzero(mask.reshape(-1), size=CAP,
                       fill_value=S * LG)[0].astype(jnp.int32)
    valid = flat < S * LG
    src = jnp.where(valid, flat // LG, S).astype(jnp.int32)
    dst = jnp.where(valid, flat - (flat // LG) * LG, LG).astype(jnp.int32)
    src3 = src.reshape(NW, CPT // GCH, GCH)
    dst3 = dst.reshape(NW, CPT // GCH, GCH)

    pad_rows = lambda x, n: jnp.pad(x, ((0, n - x.shape[0]), (0, 0)))
    surf_tab = jnp.pad(
        jnp.concatenate([jnp.pad(surface_pos, ((0, 0), (0, 13))),
                         hs0, hs1, hs2], axis=1),
        ((0, SPAD - S), (0, 48)))
    lig_tab = jnp.pad(
        jnp.concatenate([jnp.pad(init_ligand_pos, ((0, 0), (0, 13))), hl0],
                        axis=1),
        ((0, LPAD - LG), (0, 48)))

    gsurf = _sc_gather(surf_tab, src3)
    glig0 = _sc_gather(lig_tab, dst3)

    zeros = jnp.zeros((LPAD, F), jnp.float32)
    he = _edge_mlp0(gsurf, glig0, p['convs'][0], p)
    agg2 = _sc_scatter_add(he, dst3, zeros)
    hl = _lig_node(hl0, agg2, p['convs'][0])
    for li in (1, 2):
        cv = p['convs'][li]
        ghl = _sc_gather(pad_rows(hl, LPAD), dst3)
        he = _edge_mlpN(gsurf, ghl, he, cv, 16 + li * F)
        agg2 = _sc_scatter_add(he, dst3, zeros)
        hl = _lig_node(hl, agg2, cv)

    return _pos_out(hl, init_ligand_pos, p)


# speedup vs baseline: 6.9887x; 1.0549x over previous
import functools

import jax
import jax.numpy as jnp
from jax import lax
from jax.experimental import pallas as pl
from jax.experimental.pallas import tpu as pltpu
from jax.experimental.pallas import tpu_sc as plsc

S = 8000
LG = 2000
F = 64
NC, NS = 2, 16
NW = NC * NS
CPT = 5120
CAP = NW * CPT
SPAD = 8192
LPAD = 2048
EBLK = 2048
STS = 400
GCH = 128

def _mesh():
    return plsc.VectorSubcoreMesh(core_axis_name="c", subcore_axis_name="s")


def _sc_params():
    return pltpu.CompilerParams(use_tc_tiling_on_sc=False)


def _lig_prologue_body(lp_ref, t_ref, tw1, tb1, tw2, tb2, wl, bl, wg, bg, wb,
                       out_ref):
    t = t_ref[...]
    half = 32
    k = lax.broadcasted_iota(jnp.int32, (1, half), 1).astype(jnp.float32)
    freqs = jnp.exp(-jnp.log(10000.0) / (half - 1) * k)
    a = t * freqs
    ht = jnp.concatenate([jnp.sin(a), jnp.cos(a)], axis=1)
    x = jnp.dot(ht, tw1[...], preferred_element_type=jnp.float32) + tb1[...]
    c = 0.7978845608028654
    g = 0.5 * x * (1.0 + jnp.tanh(c * (x + 0.044715 * x * x * x)))
    ht = jnp.dot(g, tw2[...], preferred_element_type=jnp.float32) + tb2[...]
    lp = lp_ref[...]
    base = jnp.dot(lp, wl[...], preferred_element_type=jnp.float32) + bl[...]
    gate = jax.nn.sigmoid(
        jnp.dot(ht, wg[...], preferred_element_type=jnp.float32) + bg[...])
    out_ref[...] = base * gate + jnp.dot(ht, wb[...],
                                         preferred_element_type=jnp.float32)


def _lig_prologue(lp, t, p):
    tm, c = p['time_mlp'], p['csl']
    full = lambda s: pl.BlockSpec(s, lambda: (0,) * len(s))
    args = (lp, t,
            tm['w1'], tm['b1'].reshape(1, -1), tm['w2'], tm['b2'].reshape(1, -1),
            c['wl'], c['bl'].reshape(1, -1), c['wg'], c['bg'].reshape(1, -1),
            c['wb'])
    return pl.pallas_call(
        _lig_prologue_body,
        out_shape=jax.ShapeDtypeStruct((LG, F), jnp.float32),
        in_specs=[full(a.shape) for a in args],
        out_specs=full((LG, F)),
    )(*args)


def _ln(x, g, b):
    m = jnp.mean(x, axis=-1, keepdims=True)
    v = jnp.mean((x - m) ** 2, axis=-1, keepdims=True)
    return (x - m) * jax.lax.rsqrt(v + 1e-5) * g + b


def _surf_mask_body(sp_ref, bs_ref, lpt_ref, bl_ref, sw, sb,
                    nw1, nb1, nw2, nb2, ng, nbl,
                    mask_ref, hs0_ref, hs1_ref, hs2_ref):
    sp = sp_ref[...]
    d2 = jnp.zeros((STS, LG), jnp.float32)
    for ci in range(3):
        diff = sp[:, ci:ci + 1] - lpt_ref[ci:ci + 1, :]
        d2 = d2 + diff * diff
    same = bs_ref[...] == bl_ref[...]
    mask_ref[...] = jnp.where(same & (d2 < 9.0), jnp.int32(1), jnp.int32(0))

    hs = jnp.dot(sp, sw[...], preferred_element_type=jnp.float32) + sb[...]
    hs0_ref[...] = hs
    outs = (hs1_ref, hs2_ref)
    for li in range(2):
        up = jnp.maximum(
            jnp.dot(hs, nw1[li], preferred_element_type=jnp.float32) + nb1[li],
            0.0)
        up = jnp.dot(up, nw2[li], preferred_element_type=jnp.float32) + nb2[li]
        hs = hs + _ln(up, ng[li], nbl[li])
        outs[li][...] = hs


def _surf_mask(sp, bs, lp, bl, p):
    nw1 = jnp.stack([cv['node']['w1'][:F] for cv in p['convs'][:2]])
    nb1 = jnp.stack([cv['node']['b1'].reshape(1, -1) for cv in p['convs'][:2]])
    nw2 = jnp.stack([cv['node']['w2'] for cv in p['convs'][:2]])
    nb2 = jnp.stack([cv['node']['b2'].reshape(1, -1) for cv in p['convs'][:2]])
    ng = jnp.stack([cv['node']['ln_g'].reshape(1, -1) for cv in p['convs'][:2]])
    nbl = jnp.stack([cv['node']['ln_b'].reshape(1, -1) for cv in p['convs'][:2]])
    grid = S // STS
    tile = lambda s: pl.BlockSpec(s, lambda i: (i,) + (0,) * (len(s) - 1))
    full = lambda s: pl.BlockSpec(s, lambda i: (0,) * len(s))
    args = (sp, bs.reshape(S, 1), lp.T, bl.reshape(1, LG),
            p['surf_enc']['w'], p['surf_enc']['b'].reshape(1, -1),
            nw1, nb1, nw2, nb2, ng, nbl)
    in_specs = [tile((STS, 3)), tile((STS, 1)), full((3, LG)), full((1, LG))]
    in_specs += [full(a.shape) for a in args[4:]]
    return pl.pallas_call(
        _surf_mask_body,
        grid=(grid,),
        out_shape=[jax.ShapeDtypeStruct((S, LG), jnp.int32)] +
                  [jax.ShapeDtypeStruct((S, F), jnp.float32)] * 3,
        in_specs=in_specs,
        out_specs=[tile((STS, LG))] + [tile((STS, F))] * 3,
    )(*args)


def _sc_gather(table, idx3):
    D = table.shape[1]
    iters = CPT // GCH
    NB = max(2, min(8, 400 * 1024 // (GCH * D * 4)))

    @functools.partial(
        pl.kernel, mesh=_mesh(), compiler_params=_sc_params(),
        out_type=jax.ShapeDtypeStruct((CAP, D), jnp.float32),
        scratch_types=[pltpu.VMEM((CPT // GCH, GCH), jnp.int32),
                       pltpu.VMEM((NB, GCH, D), jnp.float32)] +
                      [pltpu.SemaphoreType.DMA] * (2 * NB))
    def k(tab_hbm, idx_hbm, out_hbm, idx_v, rows_v, *sems):
        gsem, wsem = sems[:NB], sems[NB:]
        wid = lax.axis_index("s") * NC + lax.axis_index("c")
        base = wid * CPT
        pltpu.sync_copy(idx_hbm.at[wid], idx_v)

        gh = [None] * NB
        wh = [None] * NB
        for i in range(iters + NB - 1):
            if i < iters:
                b = i % NB
                if wh[b] is not None:
                    wh[b].wait()
                gh[b] = pltpu.async_copy(
                    tab_hbm.at[idx_v.at[i]], rows_v.at[b], gsem[b])
            j = i - (NB - 1)
            if j >= 0:
                bj = j % NB
                gh[bj].wait()
                wh[bj] = pltpu.async_copy(
                    rows_v.at[bj], out_hbm.at[pl.ds(base + j * GCH, GCH)],
                    wsem[bj])
        for h in wh:
            if h is not None:
                h.wait()

    return k(table, idx3)


def _sc_scatter_add(vals, dst3, zeros):
    iters = CPT // GCH
    NB = 8

    @functools.partial(
        pl.kernel, mesh=_mesh(), compiler_params=_sc_params(),
        out_type=jax.ShapeDtypeStruct((NC, LPAD, F), jnp.float32),
        scratch_types=[pltpu.VMEM((CPT // GCH, GCH), jnp.int32),
                       pltpu.VMEM((NB, GCH, F), jnp.float32),
                       pltpu.VMEM_SHARED((LPAD, F), jnp.float32)] +
                      [pltpu.SemaphoreType.DMA] * (2 * NB))
    def k(v_hbm, d_hbm, z_hbm, out_hbm, idx_v, rows_v, acc_sh, *sems):
        lsem, asem = sems[:NB], sems[NB:]
        cid = lax.axis_index("c")
        sid = lax.axis_index("s")
        wid = sid * NC + cid
        base = wid * CPT
        stripe = LPAD // NS
        pltpu.sync_copy(z_hbm.at[pl.ds(sid * stripe, stripe)],
                        acc_sh.at[pl.ds(sid * stripe, stripe)])
        pltpu.sync_copy(d_hbm.at[wid], idx_v)
        plsc.subcore_barrier()

        lh = [None] * NB
        ah = [None] * NB
        for i in range(iters + NB - 1):
            if i < iters:
                b = i % NB
                if ah[b] is not None:
                    ah[b].wait()
                lh[b] = pltpu.async_copy(
                    v_hbm.at[pl.ds(base + i * GCH, GCH)], rows_v.at[b],
                    lsem[b])
            j = i - (NB - 1)
            if j >= 0:
                bj = j % NB
                lh[bj].wait()
                ah[bj] = pltpu.async_copy(
                    rows_v.at[bj], acc_sh.at[idx_v.at[j]], asem[bj],
                    add=True)
        for h in ah:
            if h is not None:
                h.wait()

        plsc.subcore_barrier()
        pltpu.sync_copy(acc_sh.at[pl.ds(sid * stripe, stripe)],
                        out_hbm.at[cid].at[pl.ds(sid * stripe, stripe)])

    return k(vals, dst3, zeros)


def _edge_mlp_common(gs, gl, he, w1, b1, w2, b2, g, b):
    w = w1[...]
    x = (jnp.dot(gs, w[0:F], preferred_element_type=jnp.float32) +
         jnp.dot(gl, w[F:2 * F], preferred_element_type=jnp.float32) +
         jnp.dot(he, w[2 * F:3 * F], preferred_element_type=jnp.float32)
         + b1[...])
    x = jnp.maximum(x, 0.0)
    x = jnp.dot(x, w2[...], preferred_element_type=jnp.float32) + b2[...]
    return he + _ln(x, g[...], b[...])


def _edge_mlp0_body(gsurf_ref, glig_ref, w1, b1, w2, b2, g, b, emw, off,
                    out_ref):
    ev = gsurf_ref[:, 0:3] - glig_ref[:, 0:3]
    nrm = jnp.sqrt(jnp.sum(ev * ev, axis=1, keepdims=True))
    v = ev / (nrm + 1e-7)
    coeff = -0.5 / ((10.0 / 18.0) ** 2)
    sca = jnp.exp(coeff * (nrm - off[...]) ** 2)
    em = emw[...]
    he0 = jnp.concatenate([sca] + [v[:, ci:ci + 1] * em for ci in range(3)],
                          axis=1)
    gs = gsurf_ref[:, 16:16 + F]
    gl = glig_ref[:, 16:16 + F]
    out_ref[...] = _edge_mlp_common(gs, gl, he0, w1, b1, w2, b2, g, b)


def _edge_mlp0(gsurf, glig, cv, p):
    full = lambda s: pl.BlockSpec(s, lambda i: (0,) * len(s))
    tile = lambda s: pl.BlockSpec(s, lambda i: (i,) + (0,) * (len(s) - 1))
    e = cv['edge']
    return pl.pallas_call(
        _edge_mlp0_body,
        grid=(CAP // EBLK,),
        out_shape=jax.ShapeDtypeStruct((CAP, F), jnp.float32),
        in_specs=[tile((EBLK, 208)), tile((EBLK, 80)),
                  full((192, 128)), full((1, 128)), full((128, F)),
                  full((1, F)), full((1, F)), full((1, F)),
                  full((1, 15)), full((1, 19))],
        out_specs=tile((EBLK, F)),
    )(gsurf, glig, e['w1'], e['b1'].reshape(1, -1), e['w2'],
      e['b2'].reshape(1, -1), e['ln_g'].reshape(1, -1),
      e['ln_b'].reshape(1, -1), p['edge_map_w'],
      p['gs_offset'].reshape(1, 19))


def _edge_mlpN_body(col, gsurf_ref, ghl_ref, he_ref, w1, b1, w2, b2, g, b,
                    out_ref):
    gs = gsurf_ref[:, col:col + F]
    out_ref[...] = _edge_mlp_common(gs, ghl_ref[...], he_ref[...],
                                    w1, b1, w2, b2, g, b)


def _edge_mlpN(gsurf, ghl, he, cv, col):
    full = lambda s: pl.BlockSpec(s, lambda i: (0,) * len(s))
    tile = lambda s: pl.BlockSpec(s, lambda i: (i,) + (0,) * (len(s) - 1))
    e = cv['edge']
    return pl.pallas_call(
        functools.partial(_edge_mlpN_body, col),
        grid=(CAP // EBLK,),
        out_shape=jax.ShapeDtypeStruct((CAP, F), jnp.float32),
        in_specs=[tile((EBLK, 208)), tile((EBLK, F)), tile((EBLK, F)),
                  full((192, 128)), full((1, 128)), full((128, F)),
                  full((1, F)), full((1, F)), full((1, F))],
        out_specs=tile((EBLK, F)),
    )(gsurf, ghl, he, e['w1'], e['b1'].reshape(1, -1), e['w2'],
      e['b2'].reshape(1, -1), e['ln_g'].reshape(1, -1),
      e['ln_b'].reshape(1, -1))


def _lig_node_body(hl_ref, agg_ref, w1, b1, w2, b2, g, b, out_ref):
    agg = agg_ref[0, 0:LG, :] + agg_ref[1, 0:LG, :]
    hl = hl_ref[...]
    w = w1[...]
    x = (jnp.dot(hl, w[0:F], preferred_element_type=jnp.float32) +
         jnp.dot(agg, w[F:2 * F], preferred_element_type=jnp.float32) + b1[...])
    x = jnp.maximum(x, 0.0)
    x = jnp.dot(x, w2[...], preferred_element_type=jnp.float32) + b2[...]
    out_ref[...] = hl + _ln(x, g[...], b[...])


def _lig_node(hl, agg2, cv):
    full = lambda s: pl.BlockSpec(s, lambda: (0,) * len(s))
    n = cv['node']
    return pl.pallas_call(
        _lig_node_body,
        out_shape=jax.ShapeDtypeStruct((LG, F), jnp.float32),
        in_specs=[full((LG, F)), full((NC, LPAD, F)),
                  full((128, 128)), full((1, 128)), full((128, F)),
                  full((1, F)), full((1, F)), full((1, F))],
        out_specs=full((LG, F)),
    )(hl, agg2, n['w1'], n['b1'].reshape(1, -1), n['w2'],
      n['b2'].reshape(1, -1), n['ln_g'].reshape(1, -1),
      n['ln_b'].reshape(1, -1))


def _pos_out_body(hl_ref, lp_ref, w1, b1, w2, b2, out_ref):
    x = jnp.dot(hl_ref[...], w1[...], preferred_element_type=jnp.float32) + b1[...]
    x = jnp.maximum(x, 0.0)
    x = jnp.dot(x, w2[...], preferred_element_type=jnp.float32) + b2[...]
    out_ref[...] = x + lp_ref[...]


def _pos_out(hl, lp, p):
    full = lambda s: pl.BlockSpec(s, lambda: (0,) * len(s))
    m = p['pos_mlp']
    return pl.pallas_call(
        _pos_out_body,
        out_shape=jax.ShapeDtypeStruct((LG, 3), jnp.float32),
        in_specs=[full((LG, F)), full((LG, 3)), full((F, F)), full((1, F)),
                  full((F, 3)), full((1, 3))],
        out_specs=full((LG, 3)),
    )(hl, lp, m['w1'], m['b1'].reshape(1, -1), m['w2'], m['b2'].reshape(1, -1))


def kernel(surface_pos, init_ligand_pos, batch_surface, batch_ligand, time,
           params):
    p = params
    hl0 = _lig_prologue(init_ligand_pos, time, p)
    mask, hs0, hs1, hs2 = _surf_mask(surface_pos, batch_surface,
                                     init_ligand_pos, batch_ligand, p)

    flat = jnp.nonzero(mask.reshape(-1), size=CAP,
                       fill_value=S * LG)[0].astype(jnp.int32)
    valid = flat < S * LG
    src = jnp.where(valid, flat // LG, S).astype(jnp.int32)
    dst = jnp.where(valid, flat - (flat // LG) * LG, LG).astype(jnp.int32)
    src3 = src.reshape(NW, CPT // GCH, GCH)
    dst3 = dst.reshape(NW, CPT // GCH, GCH)

    pad_rows = lambda x, n: jnp.pad(x, ((0, n - x.shape[0]), (0, 0)))
    surf_tab = jnp.pad(
        jnp.concatenate([jnp.pad(surface_pos, ((0, 0), (0, 13))),
                         hs0, hs1, hs2], axis=1),
        ((0, SPAD - S), (0, 0)))
    lig_tab = jnp.pad(
        jnp.concatenate([jnp.pad(init_ligand_pos, ((0, 0), (0, 13))), hl0],
                        axis=1),
        ((0, LPAD - LG), (0, 0)))

    gsurf = _sc_gather(surf_tab, src3)
    glig0 = _sc_gather(lig_tab, dst3)

    zeros = jnp.zeros((LPAD, F), jnp.float32)
    he = _edge_mlp0(gsurf, glig0, p['convs'][0], p)
    agg2 = _sc_scatter_add(he, dst3, zeros)
    hl = _lig_node(hl0, agg2, p['convs'][0])
    for li in (1, 2):
        cv = p['convs'][li]
        ghl = _sc_gather(pad_rows(hl, LPAD), dst3)
        he = _edge_mlpN(gsurf, ghl, he, cv, 16 + li * F)
        agg2 = _sc_scatter_add(he, dst3, zeros)
        hl = _lig_node(hl, agg2, cv)

    return _pos_out(hl, init_ligand_pos, p)


# speedup vs baseline: 8.1283x; 1.1631x over previous
import functools

import jax
import jax.numpy as jnp
from jax import lax
from jax.experimental import pallas as pl
from jax.experimental.pallas import tpu as pltpu
from jax.experimental.pallas import tpu_sc as plsc

S = 8000
LG = 2000
F = 64
NC, NS = 2, 16
NW = NC * NS
CPT = 5120
CAP = NW * CPT
SPAD = 8192
LPAD = 2048
EBLK = 2048
STS = 400
GCH = 128

def _mesh():
    return plsc.VectorSubcoreMesh(core_axis_name="c", subcore_axis_name="s")


def _sc_params():
    return pltpu.CompilerParams(use_tc_tiling_on_sc=False)


def _lig_prologue_body(lp_ref, t_ref, tw1, tb1, tw2, tb2, wl, bl, wg, bg, wb,
                       out_ref):
    t = t_ref[...]
    half = 32
    k = lax.broadcasted_iota(jnp.int32, (1, half), 1).astype(jnp.float32)
    freqs = jnp.exp(-jnp.log(10000.0) / (half - 1) * k)
    a = t * freqs
    ht = jnp.concatenate([jnp.sin(a), jnp.cos(a)], axis=1)
    x = jnp.dot(ht, tw1[...], preferred_element_type=jnp.float32) + tb1[...]
    c = 0.7978845608028654
    g = 0.5 * x * (1.0 + jnp.tanh(c * (x + 0.044715 * x * x * x)))
    ht = jnp.dot(g, tw2[...], preferred_element_type=jnp.float32) + tb2[...]
    lp = lp_ref[...]
    base = jnp.dot(lp, wl[...], preferred_element_type=jnp.float32) + bl[...]
    gate = jax.nn.sigmoid(
        jnp.dot(ht, wg[...], preferred_element_type=jnp.float32) + bg[...])
    out_ref[...] = base * gate + jnp.dot(ht, wb[...],
                                         preferred_element_type=jnp.float32)


def _lig_prologue(lp, t, p):
    tm, c = p['time_mlp'], p['csl']
    full = lambda s: pl.BlockSpec(s, lambda: (0,) * len(s))
    args = (lp, t,
            tm['w1'], tm['b1'].reshape(1, -1), tm['w2'], tm['b2'].reshape(1, -1),
            c['wl'], c['bl'].reshape(1, -1), c['wg'], c['bg'].reshape(1, -1),
            c['wb'])
    return pl.pallas_call(
        _lig_prologue_body,
        out_shape=jax.ShapeDtypeStruct((LG, F), jnp.float32),
        in_specs=[full(a.shape) for a in args],
        out_specs=full((LG, F)),
    )(*args)


def _ln(x, g, b):
    m = jnp.mean(x, axis=-1, keepdims=True)
    v = jnp.mean((x - m) ** 2, axis=-1, keepdims=True)
    return (x - m) * jax.lax.rsqrt(v + 1e-5) * g + b


def _surf_mask_body(sp_ref, bs_ref, lpt_ref, bl_ref, sw, sb,
                    nw1, nb1, nw2, nb2, ng, nbl,
                    mask_ref, hs0_ref, hs1_ref, hs2_ref):
    sp = sp_ref[...]
    d2 = jnp.zeros((STS, LG), jnp.float32)
    for ci in range(3):
        diff = sp[:, ci:ci + 1] - lpt_ref[ci:ci + 1, :]
        d2 = d2 + diff * diff
    same = bs_ref[...] == bl_ref[...]
    mask_ref[...] = jnp.where(same & (d2 < 9.0), jnp.int32(1), jnp.int32(0))

    hs = jnp.dot(sp, sw[...], preferred_element_type=jnp.float32) + sb[...]
    hs0_ref[...] = hs
    outs = (hs1_ref, hs2_ref)
    for li in range(2):
        up = jnp.maximum(
            jnp.dot(hs, nw1[li], preferred_element_type=jnp.float32) + nb1[li],
            0.0)
        up = jnp.dot(up, nw2[li], preferred_element_type=jnp.float32) + nb2[li]
        hs = hs + _ln(up, ng[li], nbl[li])
        outs[li][...] = hs


def _surf_mask(sp, bs, lp, bl, p):
    nw1 = jnp.stack([cv['node']['w1'][:F] for cv in p['convs'][:2]])
    nb1 = jnp.stack([cv['node']['b1'].reshape(1, -1) for cv in p['convs'][:2]])
    nw2 = jnp.stack([cv['node']['w2'] for cv in p['convs'][:2]])
    nb2 = jnp.stack([cv['node']['b2'].reshape(1, -1) for cv in p['convs'][:2]])
    ng = jnp.stack([cv['node']['ln_g'].reshape(1, -1) for cv in p['convs'][:2]])
    nbl = jnp.stack([cv['node']['ln_b'].reshape(1, -1) for cv in p['convs'][:2]])
    grid = S // STS
    tile = lambda s: pl.BlockSpec(s, lambda i: (i,) + (0,) * (len(s) - 1))
    full = lambda s: pl.BlockSpec(s, lambda i: (0,) * len(s))
    args = (sp, bs.reshape(S, 1), lp.T, bl.reshape(1, LG),
            p['surf_enc']['w'], p['surf_enc']['b'].reshape(1, -1),
            nw1, nb1, nw2, nb2, ng, nbl)
    in_specs = [tile((STS, 3)), tile((STS, 1)), full((3, LG)), full((1, LG))]
    in_specs += [full(a.shape) for a in args[4:]]
    return pl.pallas_call(
        _surf_mask_body,
        grid=(grid,),
        out_shape=[jax.ShapeDtypeStruct((S, LG), jnp.int32)] +
                  [jax.ShapeDtypeStruct((S, F), jnp.float32)] * 3,
        in_specs=in_specs,
        out_specs=[tile((STS, LG))] + [tile((STS, F))] * 3,
    )(*args)


def _sc_gather(table, idx3):
    D = table.shape[1]
    dt = table.dtype
    esz = table.dtype.itemsize
    iters = CPT // GCH
    NB = max(2, min(8, 400 * 1024 // (GCH * D * esz)))

    @functools.partial(
        pl.kernel, mesh=_mesh(), compiler_params=_sc_params(),
        out_type=jax.ShapeDtypeStruct((CAP, D), dt),
        scratch_types=[pltpu.VMEM((CPT // GCH, GCH), jnp.int32),
                       pltpu.VMEM((NB, GCH, D), dt)] +
                      [pltpu.SemaphoreType.DMA] * (2 * NB))
    def k(tab_hbm, idx_hbm, out_hbm, idx_v, rows_v, *sems):
        gsem, wsem = sems[:NB], sems[NB:]
        wid = lax.axis_index("s") * NC + lax.axis_index("c")
        base = wid * CPT
        pltpu.sync_copy(idx_hbm.at[wid], idx_v)

        gh = [None] * NB
        wh = [None] * NB
        for i in range(iters + NB - 1):
            if i < iters:
                b = i % NB
                if wh[b] is not None:
                    wh[b].wait()
                gh[b] = pltpu.async_copy(
                    tab_hbm.at[idx_v.at[i]], rows_v.at[b], gsem[b])
            j = i - (NB - 1)
            if j >= 0:
                bj = j % NB
                gh[bj].wait()
                wh[bj] = pltpu.async_copy(
                    rows_v.at[bj], out_hbm.at[pl.ds(base + j * GCH, GCH)],
                    wsem[bj])
        for h in wh:
            if h is not None:
                h.wait()

    return k(table, idx3)


def _sc_scatter_add(vals, dst3, zeros):
    iters = CPT // GCH
    NB = 8

    @functools.partial(
        pl.kernel, mesh=_mesh(), compiler_params=_sc_params(),
        out_type=jax.ShapeDtypeStruct((NC, LPAD, F), jnp.float32),
        scratch_types=[pltpu.VMEM((CPT // GCH, GCH), jnp.int32),
                       pltpu.VMEM((NB, GCH, F), jnp.float32),
                       pltpu.VMEM_SHARED((LPAD, F), jnp.float32)] +
                      [pltpu.SemaphoreType.DMA] * (2 * NB))
    def k(v_hbm, d_hbm, z_hbm, out_hbm, idx_v, rows_v, acc_sh, *sems):
        lsem, asem = sems[:NB], sems[NB:]
        cid = lax.axis_index("c")
        sid = lax.axis_index("s")
        wid = sid * NC + cid
        base = wid * CPT
        stripe = LPAD // NS
        pltpu.sync_copy(z_hbm.at[pl.ds(sid * stripe, stripe)],
                        acc_sh.at[pl.ds(sid * stripe, stripe)])
        pltpu.sync_copy(d_hbm.at[wid], idx_v)
        plsc.subcore_barrier()

        lh = [None] * NB
        ah = [None] * NB
        for i in range(iters + NB - 1):
            if i < iters:
                b = i % NB
                if ah[b] is not None:
                    ah[b].wait()
                lh[b] = pltpu.async_copy(
                    v_hbm.at[pl.ds(base + i * GCH, GCH)], rows_v.at[b],
                    lsem[b])
            j = i - (NB - 1)
            if j >= 0:
                bj = j % NB
                lh[bj].wait()
                ah[bj] = pltpu.async_copy(
                    rows_v.at[bj], acc_sh.at[idx_v.at[j]], asem[bj],
                    add=True)
        for h in ah:
            if h is not None:
                h.wait()

        plsc.subcore_barrier()
        pltpu.sync_copy(acc_sh.at[pl.ds(sid * stripe, stripe)],
                        out_hbm.at[cid].at[pl.ds(sid * stripe, stripe)])

    return k(vals, dst3, zeros)


def _edge_mlp_common(gs, gl, he, w1, b1, w2, b2, g, b):
    bf = jnp.bfloat16
    w = w1[...]
    x = (jnp.dot(gs, w[0:F].astype(bf), preferred_element_type=jnp.float32) +
         jnp.dot(gl, w[F:2 * F].astype(bf),
                 preferred_element_type=jnp.float32) +
         jnp.dot(he.astype(bf), w[2 * F:3 * F].astype(bf),
                 preferred_element_type=jnp.float32)
         + b1[...])
    x = jnp.maximum(x, 0.0)
    x = jnp.dot(x.astype(bf), w2[...].astype(bf),
                preferred_element_type=jnp.float32) + b2[...]
    return he + _ln(x, g[...], b[...])


def _edge_mlp0_body(gsurf_ref, glig_ref, w1, b1, w2, b2, g, b, emw, off,
                    out_ref):
    f32 = jnp.float32
    sp = gsurf_ref[:, 0:3].astype(f32) + gsurf_ref[:, 3:6].astype(f32)
    lp = glig_ref[:, 0:3].astype(f32) + glig_ref[:, 3:6].astype(f32)
    ev = sp - lp
    nrm = jnp.sqrt(jnp.sum(ev * ev, axis=1, keepdims=True))
    v = ev / (nrm + 1e-7)
    coeff = -0.5 / ((10.0 / 18.0) ** 2)
    sca = jnp.exp(coeff * (nrm - off[...]) ** 2)
    em = emw[...]
    he0 = jnp.concatenate([sca] + [v[:, ci:ci + 1] * em for ci in range(3)],
                          axis=1)
    gs = gsurf_ref[:, 16:16 + F]
    gl = glig_ref[:, 16:16 + F]
    out_ref[...] = _edge_mlp_common(gs, gl, he0, w1, b1, w2, b2, g, b)


def _edge_mlp0(gsurf, glig, cv, p):
    full = lambda s: pl.BlockSpec(s, lambda i: (0,) * len(s))
    tile = lambda s: pl.BlockSpec(s, lambda i: (i,) + (0,) * (len(s) - 1))
    e = cv['edge']
    return pl.pallas_call(
        _edge_mlp0_body,
        grid=(CAP // EBLK,),
        out_shape=jax.ShapeDtypeStruct((CAP, F), jnp.float32),
        in_specs=[tile((EBLK, 208)), tile((EBLK, 80)),
                  full((192, 128)), full((1, 128)), full((128, F)),
                  full((1, F)), full((1, F)), full((1, F)),
                  full((1, 15)), full((1, 19))],
        out_specs=tile((EBLK, F)),
    )(gsurf, glig, e['w1'], e['b1'].reshape(1, -1), e['w2'],
      e['b2'].reshape(1, -1), e['ln_g'].reshape(1, -1),
      e['ln_b'].reshape(1, -1), p['edge_map_w'],
      p['gs_offset'].reshape(1, 19))


def _edge_mlpN_body(col, gsurf_ref, ghl_ref, he_ref, w1, b1, w2, b2, g, b,
                    out_ref):
    gs = gsurf_ref[:, col:col + F]
    out_ref[...] = _edge_mlp_common(gs, ghl_ref[...], he_ref[...],
                                    w1, b1, w2, b2, g, b)


def _edge_mlpN(gsurf, ghl, he, cv, col):
    full = lambda s: pl.BlockSpec(s, lambda i: (0,) * len(s))
    tile = lambda s: pl.BlockSpec(s, lambda i: (i,) + (0,) * (len(s) - 1))
    e = cv['edge']
    return pl.pallas_call(
        functools.partial(_edge_mlpN_body, col),
        grid=(CAP // EBLK,),
        out_shape=jax.ShapeDtypeStruct((CAP, F), jnp.float32),
        in_specs=[tile((EBLK, 208)), tile((EBLK, F)), tile((EBLK, F)),
                  full((192, 128)), full((1, 128)), full((128, F)),
                  full((1, F)), full((1, F)), full((1, F))],
        out_specs=tile((EBLK, F)),
    )(gsurf, ghl, he, e['w1'], e['b1'].reshape(1, -1), e['w2'],
      e['b2'].reshape(1, -1), e['ln_g'].reshape(1, -1),
      e['ln_b'].reshape(1, -1))


def _lig_node_body(hl_ref, agg_ref, w1, b1, w2, b2, g, b, out_ref):
    agg = agg_ref[0, 0:LG, :] + agg_ref[1, 0:LG, :]
    hl = hl_ref[...]
    w = w1[...]
    x = (jnp.dot(hl, w[0:F], preferred_element_type=jnp.float32) +
         jnp.dot(agg, w[F:2 * F], preferred_element_type=jnp.float32) + b1[...])
    x = jnp.maximum(x, 0.0)
    x = jnp.dot(x, w2[...], preferred_element_type=jnp.float32) + b2[...]
    out_ref[...] = hl + _ln(x, g[...], b[...])


def _lig_node(hl, agg2, cv):
    full = lambda s: pl.BlockSpec(s, lambda: (0,) * len(s))
    n = cv['node']
    return pl.pallas_call(
        _lig_node_body,
        out_shape=jax.ShapeDtypeStruct((LG, F), jnp.float32),
        in_specs=[full((LG, F)), full((NC, LPAD, F)),
                  full((128, 128)), full((1, 128)), full((128, F)),
                  full((1, F)), full((1, F)), full((1, F))],
        out_specs=full((LG, F)),
    )(hl, agg2, n['w1'], n['b1'].reshape(1, -1), n['w2'],
      n['b2'].reshape(1, -1), n['ln_g'].reshape(1, -1),
      n['ln_b'].reshape(1, -1))


def _pos_out_body(hl_ref, lp_ref, w1, b1, w2, b2, out_ref):
    x = jnp.dot(hl_ref[...], w1[...], preferred_element_type=jnp.float32) + b1[...]
    x = jnp.maximum(x, 0.0)
    x = jnp.dot(x, w2[...], preferred_element_type=jnp.float32) + b2[...]
    out_ref[...] = x + lp_ref[...]


def _pos_out(hl, lp, p):
    full = lambda s: pl.BlockSpec(s, lambda: (0,) * len(s))
    m = p['pos_mlp']
    return pl.pallas_call(
        _pos_out_body,
        out_shape=jax.ShapeDtypeStruct((LG, 3), jnp.float32),
        in_specs=[full((LG, F)), full((LG, 3)), full((F, F)), full((1, F)),
                  full((F, 3)), full((1, 3))],
        out_specs=full((LG, 3)),
    )(hl, lp, m['w1'], m['b1'].reshape(1, -1), m['w2'], m['b2'].reshape(1, -1))


def kernel(surface_pos, init_ligand_pos, batch_surface, batch_ligand, time,
           params):
    p = params
    hl0 = _lig_prologue(init_ligand_pos, time, p)
    mask, hs0, hs1, hs2 = _surf_mask(surface_pos, batch_surface,
                                     init_ligand_pos, batch_ligand, p)

    flat = jnp.nonzero(mask.reshape(-1), size=CAP,
                       fill_value=S * LG)[0].astype(jnp.int32)
    valid = flat < S * LG
    src = jnp.where(valid, flat // LG, S).astype(jnp.int32)
    dst = jnp.where(valid, flat - (flat // LG) * LG, LG).astype(jnp.int32)
    src3 = src.reshape(NW, CPT // GCH, GCH)
    dst3 = dst.reshape(NW, CPT // GCH, GCH)

    bf = jnp.bfloat16
    pad_rows = lambda x, n: jnp.pad(x.astype(bf), ((0, n - x.shape[0]), (0, 0)))

    def poslane(pos):
        hi = pos.astype(bf)
        lo = (pos - hi.astype(jnp.float32)).astype(bf)
        return jnp.pad(jnp.concatenate([hi, lo], axis=1), ((0, 0), (0, 10)))

    surf_tab = jnp.pad(
        jnp.concatenate([poslane(surface_pos),
                         hs0.astype(bf), hs1.astype(bf), hs2.astype(bf)],
                        axis=1),
        ((0, SPAD - S), (0, 0)))
    lig_tab = jnp.pad(
        jnp.concatenate([poslane(init_ligand_pos), hl0.astype(bf)], axis=1),
        ((0, LPAD - LG), (0, 0)))

    gsurf = _sc_gather(surf_tab, src3)
    glig0 = _sc_gather(lig_tab, dst3)

    zeros = jnp.zeros((LPAD, F), jnp.float32)
    he = _edge_mlp0(gsurf, glig0, p['convs'][0], p)
    agg2 = _sc_scatter_add(he, dst3, zeros)
    hl = _lig_node(hl0, agg2, p['convs'][0])
    for li in (1, 2):
        cv = p['convs'][li]
        ghl = _sc_gather(pad_rows(hl, LPAD), dst3)
        he = _edge_mlpN(gsurf, ghl, he, cv, 16 + li * F)
        agg2 = _sc_scatter_add(he, dst3, zeros)
        hl = _lig_node(hl, agg2, cv)

    return _pos_out(hl, init_ligand_pos, p)
